# baseline (speedup 1.0000x reference)
import functools

import jax
import jax.numpy as jnp
from jax import lax
from jax.experimental import pallas as pl
from jax.experimental.pallas import tpu as pltpu
from jax.experimental.pallas import tpu_sc as plsc

N = 10000
E = 320000
D = 128
K = 1000
B = 1024

NC = 2
NS = 16
NW = NC * NS

N_PAD = 10240
E_PW = N_PAD
CH = 128
NCHUNK = E_PW // CH
CPB = 8
NBODY = NCHUNK // CPB
E_PAD = E_PW * NW
ROWS_PT = N_PAD // NS

_mesh = plsc.VectorSubcoreMesh(core_axis_name="c", subcore_axis_name="s")
_f32 = jnp.float32
_bf16 = jnp.bfloat16


def _l1_body(e_ref, w_ref, m_ref, sm_ref, nt_ref):
    e = e_ref[:]
    dn = (((1,), (1,)), ((), ()))
    sm_ref[:] = lax.dot_general(e, w_ref[:], dn, preferred_element_type=_f32)
    nt_ref[:] = lax.dot_general(e, m_ref[:], dn, preferred_element_type=_f32)


def _tc_l1(e, w, m):
    bd = 1280
    return pl.pallas_call(
        _l1_body,
        grid=(N_PAD // bd,),
        in_specs=[
            pl.BlockSpec((bd, D), lambda i: (i, 0)),
            pl.BlockSpec((D, D), lambda i: (0, 0)),
            pl.BlockSpec((D, D), lambda i: (0, 0)),
        ],
        out_specs=(pl.BlockSpec((bd, D), lambda i: (i, 0)),
                   pl.BlockSpec((bd, D), lambda i: (i, 0))),
        out_shape=(jax.ShapeDtypeStruct((N_PAD, D), _f32),
                   jax.ShapeDtypeStruct((N_PAD, D), _f32)),
    )(e, w, m)


def _softmax_rows(x):
    mx = jnp.max(x, axis=-1, keepdims=True)
    ex = jnp.exp(x - mx)
    return ex / jnp.sum(ex, axis=-1, keepdims=True)


def _mid_body(sm_ref, p_ref, u_ref, v_ref, s2_ref, n2_ref):
    h = _softmax_rows(jnp.maximum(sm_ref[:] + p_ref[0] + p_ref[1], 0.0))
    dn = (((1,), (1,)), ((), ()))
    s2_ref[:] = lax.dot_general(h, u_ref[:], dn, preferred_element_type=_f32)
    n2_ref[:] = lax.dot_general(h, v_ref[:], dn, preferred_element_type=_f32)


def _tc_mid(self_msg, p, u, v):
    bd = 1280
    grid = (N_PAD // bd,)
    return pl.pallas_call(
        _mid_body,
        grid=grid,
        in_specs=[
            pl.BlockSpec((bd, D), lambda i: (i, 0)),
            pl.BlockSpec((NC, bd, D), lambda i: (0, i, 0)),
            pl.BlockSpec((D, D), lambda i: (0, 0)),
            pl.BlockSpec((D, D), lambda i: (0, 0)),
        ],
        out_specs=(pl.BlockSpec((bd, D), lambda i: (i, 0)),
                   pl.BlockSpec((bd, D), lambda i: (i, 0))),
        out_shape=(jax.ShapeDtypeStruct((N_PAD, D), _f32),
                   jax.ShapeDtypeStruct((N_PAD, D), _f32)),
    )(self_msg, p, u, v)


def _last_body(s2_ref, p_ref, g_ref):
    g_ref[:] = _softmax_rows(jnp.maximum(s2_ref[:] + p_ref[0] + p_ref[1], 0.0))


def _tc_last(self2, p):
    bd = 1280
    return pl.pallas_call(
        _last_body,
        grid=(N_PAD // bd,),
        in_specs=[
            pl.BlockSpec((bd, D), lambda i: (i, 0)),
            pl.BlockSpec((NC, bd, D), lambda i: (0, i, 0)),
        ],
        out_specs=pl.BlockSpec((bd, D), lambda i: (i, 0)),
        out_shape=jax.ShapeDtypeStruct((N_PAD, D), _f32),
    )(self2, p)


def _head_body(a_ref, b_ref, w1_ref, b1_ref, w2_ref, b2_ref, o_ref):
    a = a_ref[:]
    b = b_ref[:]
    dn = (((1,), (1,)), ((), ()))
    z = jnp.concatenate([a * b, a + b], axis=-1)
    x = lax.dot_general(z, w1_ref[:], dn, preferred_element_type=_f32)
    x = x + b1_ref[:]
    lg = lax.dot_general(x, w2_ref[:], dn, preferred_element_type=_f32)
    lg = lg + b2_ref[:]
    col = lax.broadcasted_iota(jnp.int32, lg.shape, 1)
    lg = jnp.where(col < 2, lg, -1e30)
    o_ref[:] = _softmax_rows(lg)


def _tc_head(a, b, w1, b1r, w2p, b2p):
    return pl.pallas_call(
        _head_body,
        out_shape=jax.ShapeDtypeStruct((B, D), _f32),
    )(a, b, w1, b1r, w2p, b2p)


def _gather1_body(tab, idxr, out_e, idx_v, rows_v, sem_a, sem_b):
    c = lax.axis_index("c")
    s = lax.axis_index("s")
    w = c * NS + s
    npw = N_PAD // NW
    pltpu.sync_copy(idxr.at[w], idx_v)
    d0 = pltpu.async_copy(tab.at[idx_v.at[0]], rows_v.at[pl.ds(0, 80)],
                          sem_a)
    d1 = pltpu.async_copy(tab.at[idx_v.at[1]], rows_v.at[pl.ds(80, 80)],
                          sem_b)
    d0.wait()
    d2 = pltpu.async_copy(tab.at[idx_v.at[2]], rows_v.at[pl.ds(160, 80)],
                          sem_a)
    d1.wait()
    d3 = pltpu.async_copy(tab.at[idx_v.at[3]], rows_v.at[pl.ds(240, 80)],
                          sem_b)
    d2.wait()
    d3.wait()
    pltpu.sync_copy(rows_v, out_e.at[pl.ds(w * npw, npw)])


def _sc_gather1(tab, idx_r):
    npw = N_PAD // NW
    return pl.kernel(
        _gather1_body,
        out_type=jax.ShapeDtypeStruct((N_PAD, D), _f32),
        mesh=_mesh,
        scratch_types=[
            pltpu.VMEM((4, 80), jnp.int32),
            pltpu.VMEM((npw, D), _f32),
            pltpu.SemaphoreType.DMA,
            pltpu.SemaphoreType.DMA,
        ],
    )(tab, idx_r)


def _segsum_body(table, idxr, zeros, out, ib, ra, rb,
                 acc, ga, gb, sa, sb):
    c = lax.axis_index("c")
    s = lax.axis_index("s")
    w = c * NS + s
    pltpu.sync_copy(zeros.at[pl.ds(s * ROWS_PT, ROWS_PT)],
                    acc.at[pl.ds(s * ROWS_PT, ROWS_PT)])
    plsc.subcore_barrier()

    def body(i, carry):
        pltpu.sync_copy(idxr.at[w].at[pl.ds(i * CPB, CPB)], ib)
        prev = None
        for j in range(0, CPB, 2):
            if prev is not None:
                prev[0].wait()
            g0 = pltpu.async_copy(table.at[ib.at[j].at[0]], ra, ga)
            if prev is not None:
                prev[1].wait()
            g1 = pltpu.async_copy(table.at[ib.at[j + 1].at[0]], rb, gb)
            g0.wait()
            s0 = pltpu.async_copy(ra, acc.at[ib.at[j].at[1]], sa, add=True)
            g1.wait()
            s1 = pltpu.async_copy(rb, acc.at[ib.at[j + 1].at[1]], sb,
                                  add=True)
            prev = (s0, s1)
        prev[0].wait()
        prev[1].wait()
        return carry

    lax.fori_loop(0, NBODY, body, 0)
    plsc.subcore_barrier()
    pltpu.sync_copy(acc.at[pl.ds(s * ROWS_PT, ROWS_PT)],
                    out.at[c].at[pl.ds(s * ROWS_PT, ROWS_PT)])


def _sc_segsum(table, idx_r, zeros):
    return pl.kernel(
        _segsum_body,
        out_type=jax.ShapeDtypeStruct((NC, N_PAD, D), _f32),
        mesh=_mesh,
        scratch_types=[
            pltpu.VMEM((CPB, 2, CH), jnp.int32),
            pltpu.VMEM((CH, D), _f32),
            pltpu.VMEM((CH, D), _f32),
            pltpu.VMEM_SHARED((N_PAD, D), _f32),
            pltpu.SemaphoreType.DMA,
            pltpu.SemaphoreType.DMA,
            pltpu.SemaphoreType.DMA,
            pltpu.SemaphoreType.DMA,
        ],
    )(table, idx_r, zeros)


def _pairs_body(g, bir, bjr, outa, outb, bi_v, bj_v, arows, brows, sem):
    c = lax.axis_index("c")
    s = lax.axis_index("s")
    w = c * NS + s
    bpw = B // NW
    pltpu.sync_copy(bir.at[w], bi_v)
    pltpu.sync_copy(bjr.at[w], bj_v)
    pltpu.async_copy(g.at[bi_v], arows, sem).wait()
    pltpu.async_copy(g.at[bj_v], brows, sem).wait()
    pltpu.sync_copy(arows, outa.at[pl.ds(w * bpw, bpw)])
    pltpu.sync_copy(brows, outb.at[pl.ds(w * bpw, bpw)])


def _sc_pairs(g, bi_r, bj_r):
    bpw = B // NW
    return pl.kernel(
        _pairs_body,
        out_type=(jax.ShapeDtypeStruct((B, D), _f32),
                  jax.ShapeDtypeStruct((B, D), _f32)),
        mesh=_mesh,
        scratch_types=[
            pltpu.VMEM((bpw,), jnp.int32),
            pltpu.VMEM((bpw,), jnp.int32),
            pltpu.VMEM((bpw, D), _f32),
            pltpu.VMEM((bpw, D), _f32),
            pltpu.SemaphoreType.DMA,
        ],
    )(g, bi_r, bj_r)


def kernel(batch, node_internal_ids, edge_index, Emb, W, M, U, V, W1, b1,
           W2, b2):
    ids = node_internal_ids.astype(jnp.int32)
    src = edge_index[0].astype(jnp.int32)
    dst = edge_index[1].astype(jnp.int32)
    bi = batch[:, 0].astype(jnp.int32)
    bj = batch[:, 1].astype(jnp.int32)

    pad_n = N_PAD - N
    ids_r = jnp.concatenate(
        [ids, (jnp.arange(pad_n, dtype=jnp.int32) % K)]).reshape(NW, 4, 80)
    pad_e = E_PAD - E
    pad_src = (jnp.arange(pad_e, dtype=jnp.int32) * 7) % N
    pad_dst = N + (jnp.arange(pad_e, dtype=jnp.int32) % 16)
    src_r = jnp.concatenate([src, pad_src]).reshape(NW, NCHUNK, 1, CH)
    dst_r = jnp.concatenate([dst, pad_dst]).reshape(NW, NCHUNK, 1, CH)
    idx_r = jnp.concatenate([src_r, dst_r], axis=2)
    zeros = jnp.zeros((N_PAD, D), _f32)
    bi_r = bi.reshape(NW, B // NW)
    bj_r = bj.reshape(NW, B // NW)
    b1r = b1.reshape(1, D)
    w2p = jnp.pad(W2, ((0, D - 2), (0, 0)))
    b2p = jnp.pad(b2, (0, D - 2)).reshape(1, D)

    e = _sc_gather1(Emb, ids_r)
    self_msg, nbr_t = _tc_l1(e, W, M)
    p1 = _sc_segsum(nbr_t, idx_r, zeros)
    self2, nbr2 = _tc_mid(self_msg, p1, U, V)
    p2 = _sc_segsum(nbr2, idx_r, zeros)
    g = _tc_last(self2, p2)
    a_rows, b_rows = _sc_pairs(g, bi_r, bj_r)
    out = _tc_head(a_rows, b_rows, W1, b1r, w2p, b2p)
    return out[:, :2]


# speedup vs baseline: 1.0194x; 1.0194x over previous
import functools

import jax
import jax.numpy as jnp
from jax import lax
from jax.experimental import pallas as pl
from jax.experimental.pallas import tpu as pltpu
from jax.experimental.pallas import tpu_sc as plsc

N = 10000
E = 320000
D = 128
K = 1000
B = 1024

NC = 2
NS = 16
NW = NC * NS

N_PAD = 10240
E_PW = N_PAD
CH = 128
NCHUNK = E_PW // CH
CPB = 8
NBODY = NCHUNK // CPB
E_PAD = E_PW * NW
ROWS_PT = N_PAD // NS

_mesh = plsc.VectorSubcoreMesh(core_axis_name="c", subcore_axis_name="s")
_f32 = jnp.float32
_bf16 = jnp.bfloat16


def _mm2_body(emb_ref, w_ref, m_ref, ew_ref, em_ref):
    e = emb_ref[:]
    dn = (((1,), (1,)), ((), ()))
    ew_ref[:] = lax.dot_general(e, w_ref[:], dn, preferred_element_type=_f32)
    em_ref[:] = lax.dot_general(e, m_ref[:], dn, preferred_element_type=_f32)


def _tc_mm2(emb, w, m):
    return pl.pallas_call(
        _mm2_body,
        out_shape=(jax.ShapeDtypeStruct((K, D), _f32),
                   jax.ShapeDtypeStruct((K, D), _f32)),
    )(emb, w, m)


def _softmax_rows(x):
    mx = jnp.max(x, axis=-1, keepdims=True)
    ex = jnp.exp(x - mx)
    return ex / jnp.sum(ex, axis=-1, keepdims=True)


def _mid_body(sm_ref, p_ref, u_ref, v_ref, s2_ref, n2_ref):
    h = _softmax_rows(jnp.maximum(sm_ref[:] + p_ref[0] + p_ref[1], 0.0))
    dn = (((1,), (1,)), ((), ()))
    s2_ref[:] = lax.dot_general(h, u_ref[:], dn, preferred_element_type=_f32)
    n2_ref[:] = lax.dot_general(h, v_ref[:], dn, preferred_element_type=_f32)


def _tc_mid(self_msg, p, u, v):
    bd = 1280
    grid = (N_PAD // bd,)
    return pl.pallas_call(
        _mid_body,
        grid=grid,
        in_specs=[
            pl.BlockSpec((bd, D), lambda i: (i, 0)),
            pl.BlockSpec((NC, bd, D), lambda i: (0, i, 0)),
            pl.BlockSpec((D, D), lambda i: (0, 0)),
            pl.BlockSpec((D, D), lambda i: (0, 0)),
        ],
        out_specs=(pl.BlockSpec((bd, D), lambda i: (i, 0)),
                   pl.BlockSpec((bd, D), lambda i: (i, 0))),
        out_shape=(jax.ShapeDtypeStruct((N_PAD, D), _f32),
                   jax.ShapeDtypeStruct((N_PAD, D), _f32)),
    )(self_msg, p, u, v)


def _last_body(s2_ref, p_ref, g_ref):
    g_ref[:] = _softmax_rows(jnp.maximum(s2_ref[:] + p_ref[0] + p_ref[1], 0.0))


def _tc_last(self2, p):
    bd = 1280
    return pl.pallas_call(
        _last_body,
        grid=(N_PAD // bd,),
        in_specs=[
            pl.BlockSpec((bd, D), lambda i: (i, 0)),
            pl.BlockSpec((NC, bd, D), lambda i: (0, i, 0)),
        ],
        out_specs=pl.BlockSpec((bd, D), lambda i: (i, 0)),
        out_shape=jax.ShapeDtypeStruct((N_PAD, D), _f32),
    )(self2, p)


def _head_body(a_ref, b_ref, w1_ref, b1_ref, w2_ref, b2_ref, o_ref):
    a = a_ref[:]
    b = b_ref[:]
    dn = (((1,), (1,)), ((), ()))
    z = jnp.concatenate([a * b, a + b], axis=-1)
    x = lax.dot_general(z, w1_ref[:], dn, preferred_element_type=_f32)
    x = x + b1_ref[:]
    lg = lax.dot_general(x, w2_ref[:], dn, preferred_element_type=_f32)
    lg = lg + b2_ref[:]
    col = lax.broadcasted_iota(jnp.int32, lg.shape, 1)
    lg = jnp.where(col < 2, lg, -1e30)
    o_ref[:] = _softmax_rows(lg)


def _tc_head(a, b, w1, b1r, w2p, b2p):
    return pl.pallas_call(
        _head_body,
        out_shape=jax.ShapeDtypeStruct((B, D), _f32),
    )(a, b, w1, b1r, w2p, b2p)


def _gather2_body(tw, tm, idxr, out_w, out_m, idx_v, rows_w, rows_m,
                  sem_w, sem_m):
    c = lax.axis_index("c")
    s = lax.axis_index("s")
    w = c * NS + s
    npw = N_PAD // NW
    pltpu.sync_copy(idxr.at[w], idx_v)
    for ch in range(4):
        dw = pltpu.async_copy(tw.at[idx_v.at[ch]],
                              rows_w.at[pl.ds(ch * 80, 80)], sem_w)
        dm = pltpu.async_copy(tm.at[idx_v.at[ch]],
                              rows_m.at[pl.ds(ch * 80, 80)], sem_m)
        dw.wait()
        dm.wait()
    pltpu.sync_copy(rows_w, out_w.at[pl.ds(w * npw, npw)])
    pltpu.sync_copy(rows_m, out_m.at[pl.ds(w * npw, npw)])


def _sc_gather2(emb_w, emb_m, idx_r):
    npw = N_PAD // NW
    return pl.kernel(
        _gather2_body,
        out_type=(jax.ShapeDtypeStruct((N_PAD, D), _f32),
                  jax.ShapeDtypeStruct((N_PAD, D), _f32)),
        mesh=_mesh,
        scratch_types=[
            pltpu.VMEM((4, 80), jnp.int32),
            pltpu.VMEM((npw, D), _f32),
            pltpu.VMEM((npw, D), _f32),
            pltpu.SemaphoreType.DMA,
            pltpu.SemaphoreType.DMA,
        ],
    )(emb_w, emb_m, idx_r)


def _segsum_body(table, idxr, zeros, out, ib, ra, rb,
                 acc, ga, gb, sa, sb):
    c = lax.axis_index("c")
    s = lax.axis_index("s")
    w = c * NS + s
    pltpu.sync_copy(zeros.at[pl.ds(s * ROWS_PT, ROWS_PT)],
                    acc.at[pl.ds(s * ROWS_PT, ROWS_PT)])
    plsc.subcore_barrier()

    def body(i, carry):
        pltpu.sync_copy(idxr.at[w].at[pl.ds(i * CPB, CPB)], ib)
        prev = None
        for j in range(0, CPB, 2):
            if prev is not None:
                prev[0].wait()
            g0 = pltpu.async_copy(table.at[ib.at[j].at[0]], ra, ga)
            if prev is not None:
                prev[1].wait()
            g1 = pltpu.async_copy(table.at[ib.at[j + 1].at[0]], rb, gb)
            g0.wait()
            s0 = pltpu.async_copy(ra, acc.at[ib.at[j].at[1]], sa, add=True)
            g1.wait()
            s1 = pltpu.async_copy(rb, acc.at[ib.at[j + 1].at[1]], sb,
                                  add=True)
            prev = (s0, s1)
        prev[0].wait()
        prev[1].wait()
        return carry

    lax.fori_loop(0, NBODY, body, 0)
    plsc.subcore_barrier()
    pltpu.sync_copy(acc.at[pl.ds(s * ROWS_PT, ROWS_PT)],
                    out.at[c].at[pl.ds(s * ROWS_PT, ROWS_PT)])


def _sc_segsum(table, idx_r, zeros):
    return pl.kernel(
        _segsum_body,
        out_type=jax.ShapeDtypeStruct((NC, N_PAD, D), _f32),
        mesh=_mesh,
        scratch_types=[
            pltpu.VMEM((CPB, 2, CH), jnp.int32),
            pltpu.VMEM((CH, D), _f32),
            pltpu.VMEM((CH, D), _f32),
            pltpu.VMEM_SHARED((N_PAD, D), _f32),
            pltpu.SemaphoreType.DMA,
            pltpu.SemaphoreType.DMA,
            pltpu.SemaphoreType.DMA,
            pltpu.SemaphoreType.DMA,
        ],
    )(table, idx_r, zeros)


def _pairs_body(g, bir, bjr, outa, outb, bi_v, bj_v, arows, brows, sem):
    c = lax.axis_index("c")
    s = lax.axis_index("s")
    w = c * NS + s
    bpw = B // NW
    pltpu.sync_copy(bir.at[w], bi_v)
    pltpu.sync_copy(bjr.at[w], bj_v)
    pltpu.async_copy(g.at[bi_v], arows, sem).wait()
    pltpu.async_copy(g.at[bj_v], brows, sem).wait()
    pltpu.sync_copy(arows, outa.at[pl.ds(w * bpw, bpw)])
    pltpu.sync_copy(brows, outb.at[pl.ds(w * bpw, bpw)])


def _sc_pairs(g, bi_r, bj_r):
    bpw = B // NW
    return pl.kernel(
        _pairs_body,
        out_type=(jax.ShapeDtypeStruct((B, D), _f32),
                  jax.ShapeDtypeStruct((B, D), _f32)),
        mesh=_mesh,
        scratch_types=[
            pltpu.VMEM((bpw,), jnp.int32),
            pltpu.VMEM((bpw,), jnp.int32),
            pltpu.VMEM((bpw, D), _f32),
            pltpu.VMEM((bpw, D), _f32),
            pltpu.SemaphoreType.DMA,
        ],
    )(g, bi_r, bj_r)


def kernel(batch, node_internal_ids, edge_index, Emb, W, M, U, V, W1, b1,
           W2, b2):
    ids = node_internal_ids.astype(jnp.int32)
    src = edge_index[0].astype(jnp.int32)
    dst = edge_index[1].astype(jnp.int32)
    bi = batch[:, 0].astype(jnp.int32)
    bj = batch[:, 1].astype(jnp.int32)

    pad_n = N_PAD - N
    ids_r = jnp.concatenate(
        [ids, (jnp.arange(pad_n, dtype=jnp.int32) % K)]).reshape(NW, 4, 80)
    pad_e = E_PAD - E
    pad_src = (jnp.arange(pad_e, dtype=jnp.int32) * 7) % N
    pad_dst = N + (jnp.arange(pad_e, dtype=jnp.int32) % 16)
    src_r = jnp.concatenate([src, pad_src]).reshape(NW, NCHUNK, 1, CH)
    dst_r = jnp.concatenate([dst, pad_dst]).reshape(NW, NCHUNK, 1, CH)
    idx_r = jnp.concatenate([src_r, dst_r], axis=2)
    zeros = jnp.zeros((N_PAD, D), _f32)
    bi_r = bi.reshape(NW, B // NW)
    bj_r = bj.reshape(NW, B // NW)
    b1r = b1.reshape(1, D)
    w2p = jnp.pad(W2, ((0, D - 2), (0, 0)))
    b2p = jnp.pad(b2, (0, D - 2)).reshape(1, D)

    emb_w, emb_m = _tc_mm2(Emb, W, M)
    self_msg, nbr_t = _sc_gather2(emb_w, emb_m, ids_r)
    p1 = _sc_segsum(nbr_t, idx_r, zeros)
    self2, nbr2 = _tc_mid(self_msg, p1, U, V)
    p2 = _sc_segsum(nbr2, idx_r, zeros)
    g = _tc_last(self2, p2)
    a_rows, b_rows = _sc_pairs(g, bi_r, bj_r)
    out = _tc_head(a_rows, b_rows, W1, b1r, w2p, b2p)
    return out[:, :2]


# speedup vs baseline: 1.0491x; 1.0291x over previous
import functools

import jax
import jax.numpy as jnp
from jax import lax
from jax.experimental import pallas as pl
from jax.experimental.pallas import tpu as pltpu
from jax.experimental.pallas import tpu_sc as plsc

N = 10000
E = 320000
D = 128
K = 1000
B = 1024

NC = 2
NS = 16
NW = NC * NS

N_PAD = 10240
E_PW = N_PAD
CH = 128
NCHUNK = E_PW // CH
CPB = 8
NBODY = NCHUNK // CPB
E_PAD = E_PW * NW
ROWS_PT = N_PAD // NS

_mesh = plsc.VectorSubcoreMesh(core_axis_name="c", subcore_axis_name="s")
_f32 = jnp.float32
_bf16 = jnp.bfloat16


def _mm2_body(emb_ref, w_ref, m_ref, ew_ref, em_ref):
    e = emb_ref[:]
    dn = (((1,), (1,)), ((), ()))
    ew_ref[:] = lax.dot_general(e, w_ref[:], dn, preferred_element_type=_f32)
    em_ref[:] = lax.dot_general(e, m_ref[:], dn, preferred_element_type=_f32)


def _tc_mm2(emb, w, m):
    return pl.pallas_call(
        _mm2_body,
        out_shape=(jax.ShapeDtypeStruct((K, D), _f32),
                   jax.ShapeDtypeStruct((K, D), _f32)),
    )(emb, w, m)


def _softmax_rows(x):
    mx = jnp.max(x, axis=-1, keepdims=True)
    ex = jnp.exp(x - mx)
    return ex / jnp.sum(ex, axis=-1, keepdims=True)


def _mid_body(sm_ref, p_ref, u_ref, v_ref, s2_ref, n2_ref):
    h = _softmax_rows(jnp.maximum(sm_ref[:] + p_ref[0] + p_ref[1], 0.0))
    dn = (((1,), (1,)), ((), ()))
    s2_ref[:] = lax.dot_general(h, u_ref[:], dn, preferred_element_type=_f32)
    n2_ref[:] = lax.dot_general(h, v_ref[:], dn, preferred_element_type=_f32)


def _tc_mid(self_msg, p, u, v):
    bd = 1280
    grid = (N_PAD // bd,)
    return pl.pallas_call(
        _mid_body,
        grid=grid,
        in_specs=[
            pl.BlockSpec((bd, D), lambda i: (i, 0)),
            pl.BlockSpec((NC, bd, D), lambda i: (0, i, 0)),
            pl.BlockSpec((D, D), lambda i: (0, 0)),
            pl.BlockSpec((D, D), lambda i: (0, 0)),
        ],
        out_specs=(pl.BlockSpec((bd, D), lambda i: (i, 0)),
                   pl.BlockSpec((bd, D), lambda i: (i, 0))),
        out_shape=(jax.ShapeDtypeStruct((N_PAD, D), _f32),
                   jax.ShapeDtypeStruct((N_PAD, D), _f32)),
    )(self_msg, p, u, v)


def _last_body(s2_ref, p_ref, g_ref):
    g_ref[:] = _softmax_rows(jnp.maximum(s2_ref[:] + p_ref[0] + p_ref[1], 0.0))


def _tc_last(self2, p):
    bd = 1280
    return pl.pallas_call(
        _last_body,
        grid=(N_PAD // bd,),
        in_specs=[
            pl.BlockSpec((bd, D), lambda i: (i, 0)),
            pl.BlockSpec((NC, bd, D), lambda i: (0, i, 0)),
        ],
        out_specs=pl.BlockSpec((bd, D), lambda i: (i, 0)),
        out_shape=jax.ShapeDtypeStruct((N_PAD, D), _f32),
    )(self2, p)


def _head_body(a_ref, b_ref, w1_ref, b1_ref, w2_ref, b2_ref, o_ref):
    a = a_ref[:]
    b = b_ref[:]
    dn = (((1,), (1,)), ((), ()))
    z = jnp.concatenate([a * b, a + b], axis=-1)
    x = lax.dot_general(z, w1_ref[:], dn, preferred_element_type=_f32)
    x = x + b1_ref[:]
    lg = lax.dot_general(x, w2_ref[:], dn, preferred_element_type=_f32)
    lg = lg + b2_ref[:]
    col = lax.broadcasted_iota(jnp.int32, lg.shape, 1)
    lg = jnp.where(col < 2, lg, -1e30)
    o_ref[:] = _softmax_rows(lg)


def _tc_head(a, b, w1, b1r, w2p, b2p):
    return pl.pallas_call(
        _head_body,
        out_shape=jax.ShapeDtypeStruct((B, D), _f32),
    )(a, b, w1, b1r, w2p, b2p)


def _gather2_body(tw, tm, idxr, out_w, out_m, idx_v, rows_w, rows_m,
                  sem_w, sem_m):
    c = lax.axis_index("c")
    s = lax.axis_index("s")
    w = c * NS + s
    npw = N_PAD // NW
    pltpu.sync_copy(idxr.at[w], idx_v)
    for ch in range(4):
        dw = pltpu.async_copy(tw.at[idx_v.at[ch]],
                              rows_w.at[pl.ds(ch * 80, 80)], sem_w)
        dm = pltpu.async_copy(tm.at[idx_v.at[ch]],
                              rows_m.at[pl.ds(ch * 80, 80)], sem_m)
        dw.wait()
        dm.wait()
    pltpu.sync_copy(rows_w, out_w.at[pl.ds(w * npw, npw)])
    pltpu.sync_copy(rows_m, out_m.at[pl.ds(w * npw, npw)])


def _sc_gather2(emb_w, emb_m, idx_r):
    npw = N_PAD // NW
    return pl.kernel(
        _gather2_body,
        out_type=(jax.ShapeDtypeStruct((N_PAD, D), _f32),
                  jax.ShapeDtypeStruct((N_PAD, D), _f32)),
        mesh=_mesh,
        scratch_types=[
            pltpu.VMEM((4, 80), jnp.int32),
            pltpu.VMEM((npw, D), _f32),
            pltpu.VMEM((npw, D), _f32),
            pltpu.SemaphoreType.DMA,
            pltpu.SemaphoreType.DMA,
        ],
    )(emb_w, emb_m, idx_r)


def _segsum_body(table, idxr, zeros, out, ib, ra, rb,
                 acc, ga, gb, sa, sb, ia):
    c = lax.axis_index("c")
    s = lax.axis_index("s")
    w = c * NS + s
    pltpu.sync_copy(zeros.at[pl.ds(s * ROWS_PT, ROWS_PT)],
                    acc.at[pl.ds(s * ROWS_PT, ROWS_PT)])
    plsc.subcore_barrier()

    pltpu.async_copy(idxr.at[w].at[pl.ds(0, CPB)], ib.at[0], ia)

    def body(i, carry):
        h = lax.rem(i, 2)
        pltpu.make_async_copy(idxr.at[w].at[pl.ds(i * CPB, CPB)],
                              ib.at[h], ia).wait()
        pltpu.async_copy(idxr.at[w].at[pl.ds((i + 1) * CPB, CPB)],
                         ib.at[1 - h], ia)
        ibh = ib.at[h]
        prev = None
        for j in range(0, CPB, 2):
            if prev is not None:
                prev[0].wait()
            g0 = pltpu.async_copy(table.at[ibh.at[j].at[0]], ra, ga)
            if prev is not None:
                prev[1].wait()
            g1 = pltpu.async_copy(table.at[ibh.at[j + 1].at[0]], rb, gb)
            g0.wait()
            s0 = pltpu.async_copy(ra, acc.at[ibh.at[j].at[1]], sa, add=True)
            g1.wait()
            s1 = pltpu.async_copy(rb, acc.at[ibh.at[j + 1].at[1]], sb,
                                  add=True)
            prev = (s0, s1)
        prev[0].wait()
        prev[1].wait()
        return carry

    lax.fori_loop(0, NBODY, body, 0)
    pltpu.make_async_copy(idxr.at[w].at[pl.ds(0, CPB)],
                          ib.at[0], ia).wait()
    plsc.subcore_barrier()
    pltpu.sync_copy(acc.at[pl.ds(s * ROWS_PT, ROWS_PT)],
                    out.at[c].at[pl.ds(s * ROWS_PT, ROWS_PT)])


def _sc_segsum(table, idx_r, zeros):
    return pl.kernel(
        _segsum_body,
        out_type=jax.ShapeDtypeStruct((NC, N_PAD, D), _f32),
        mesh=_mesh,
        scratch_types=[
            pltpu.VMEM((2, CPB, 2, CH), jnp.int32),
            pltpu.VMEM((CH, D), _f32),
            pltpu.VMEM((CH, D), _f32),
            pltpu.VMEM_SHARED((N_PAD, D), _f32),
            pltpu.SemaphoreType.DMA,
            pltpu.SemaphoreType.DMA,
            pltpu.SemaphoreType.DMA,
            pltpu.SemaphoreType.DMA,
            pltpu.SemaphoreType.DMA,
        ],
    )(table, idx_r, zeros)


def _pairs_body(g, bir, bjr, outa, outb, bi_v, bj_v, arows, brows, sem):
    c = lax.axis_index("c")
    s = lax.axis_index("s")
    w = c * NS + s
    bpw = B // NW
    pltpu.sync_copy(bir.at[w], bi_v)
    pltpu.sync_copy(bjr.at[w], bj_v)
    pltpu.async_copy(g.at[bi_v], arows, sem).wait()
    pltpu.async_copy(g.at[bj_v], brows, sem).wait()
    pltpu.sync_copy(arows, outa.at[pl.ds(w * bpw, bpw)])
    pltpu.sync_copy(brows, outb.at[pl.ds(w * bpw, bpw)])


def _sc_pairs(g, bi_r, bj_r):
    bpw = B // NW
    return pl.kernel(
        _pairs_body,
        out_type=(jax.ShapeDtypeStruct((B, D), _f32),
                  jax.ShapeDtypeStruct((B, D), _f32)),
        mesh=_mesh,
        scratch_types=[
            pltpu.VMEM((bpw,), jnp.int32),
            pltpu.VMEM((bpw,), jnp.int32),
            pltpu.VMEM((bpw, D), _f32),
            pltpu.VMEM((bpw, D), _f32),
            pltpu.SemaphoreType.DMA,
        ],
    )(g, bi_r, bj_r)


def kernel(batch, node_internal_ids, edge_index, Emb, W, M, U, V, W1, b1,
           W2, b2):
    ids = node_internal_ids.astype(jnp.int32)
    src = edge_index[0].astype(jnp.int32)
    dst = edge_index[1].astype(jnp.int32)
    bi = batch[:, 0].astype(jnp.int32)
    bj = batch[:, 1].astype(jnp.int32)

    pad_n = N_PAD - N
    ids_r = jnp.concatenate(
        [ids, (jnp.arange(pad_n, dtype=jnp.int32) % K)]).reshape(NW, 4, 80)
    pad_e = E_PAD - E
    pad_src = (jnp.arange(pad_e, dtype=jnp.int32) * 7) % N
    pad_dst = N + (jnp.arange(pad_e, dtype=jnp.int32) % 16)
    src_r = jnp.concatenate([src, pad_src]).reshape(NW, NCHUNK, 1, CH)
    dst_r = jnp.concatenate([dst, pad_dst]).reshape(NW, NCHUNK, 1, CH)
    idx_r = jnp.concatenate([src_r, dst_r], axis=2)
    idx_r = jnp.concatenate(
        [idx_r, jnp.zeros((NW, CPB, 2, CH), jnp.int32)], axis=1)
    zeros = jnp.zeros((N_PAD, D), _f32)
    bi_r = bi.reshape(NW, B // NW)
    bj_r = bj.reshape(NW, B // NW)
    b1r = b1.reshape(1, D)
    w2p = jnp.pad(W2, ((0, D - 2), (0, 0)))
    b2p = jnp.pad(b2, (0, D - 2)).reshape(1, D)

    emb_w, emb_m = _tc_mm2(Emb, W, M)
    self_msg, nbr_t = _sc_gather2(emb_w, emb_m, ids_r)
    p1 = _sc_segsum(nbr_t, idx_r, zeros)
    self2, nbr2 = _tc_mid(self_msg, p1, U, V)
    p2 = _sc_segsum(nbr2, idx_r, zeros)
    g = _tc_last(self2, p2)
    a_rows, b_rows = _sc_pairs(g, bi_r, bj_r)
    out = _tc_head(a_rows, b_rows, W1, b1r, w2p, b2p)
    return out[:, :2]
